# baseline (speedup 1.0000x reference)
import functools

import jax
import jax.numpy as jnp
from jax import lax
from jax.experimental import pallas as pl
from jax.experimental.pallas import tpu as pltpu
from jax.experimental.pallas import tpu_sc as plsc

B = 16384
D = 64
W = 2 * D
NC = 2
NS = 16
NW = NC * NS
BPW = B // NW
CH = 128
NCH = BPW // CH
EPS = 1e-12
TBLK = 1024

_mesh = plsc.VectorSubcoreMesh(core_axis_name="c", subcore_axis_name="s")


@functools.partial(
    pl.kernel,
    mesh=_mesh,
    compiler_params=pltpu.CompilerParams(use_tc_tiling_on_sc=True),
    out_type=jax.ShapeDtypeStruct((B, W), jnp.float32),
    scratch_types=[
        pltpu.VMEM((BPW,), jnp.int32),
        pltpu.VMEM((BPW, W), jnp.float32),
        pltpu.SemaphoreType.DMA,
    ],
)
def _gather(idx_hbm, tab_hbm, out_hbm, idx_v, rows_v, sem):
    wid = lax.axis_index("s") * NC + lax.axis_index("c")
    base = wid * BPW

    pltpu.sync_copy(idx_hbm.at[pl.ds(base, BPW)], idx_v)

    copies = []
    for j in range(NCH):
        copies.append(pltpu.async_copy(
            tab_hbm.at[idx_v.at[pl.ds(j * CH, CH)]],
            rows_v.at[pl.ds(j * CH, CH)], sem))
    for c in copies:
        c.wait()

    pltpu.sync_copy(rows_v, out_hbm.at[pl.ds(base, BPW)])


N = 100000
CBLK = 5000


def _concat_body(r_ref, i_ref, o_ref):
    o_ref[:, :D] = r_ref[...]
    o_ref[:, D:] = i_ref[...]


_concat = pl.pallas_call(
    _concat_body,
    grid=(N // CBLK,),
    in_specs=[pl.BlockSpec((CBLK, D), lambda i: (i, 0)),
              pl.BlockSpec((CBLK, D), lambda i: (i, 0))],
    out_specs=pl.BlockSpec((CBLK, W), lambda i: (i, 0)),
    out_shape=jax.ShapeDtypeStruct((N, W), jnp.float32),
)


def _normalize_body(x_ref, or_ref, oi_ref):
    x = x_ref[...]
    r = x[:, :D]
    m = x[:, D:]
    s = jnp.sum(r * r + m * m, axis=1, keepdims=True) + jnp.float32(EPS)
    inv = lax.rsqrt(s)
    or_ref[...] = r * inv
    oi_ref[...] = m * inv


_normalize = pl.pallas_call(
    _normalize_body,
    grid=(B // TBLK,),
    in_specs=[pl.BlockSpec((TBLK, W), lambda i: (i, 0))],
    out_specs=[pl.BlockSpec((TBLK, D), lambda i: (i, 0)),
               pl.BlockSpec((TBLK, D), lambda i: (i, 0))],
    out_shape=[jax.ShapeDtypeStruct((B, D), jnp.float32),
               jax.ShapeDtypeStruct((B, D), jnp.float32)],
)


def kernel(subcarrier_indices, basis_real, basis_imag):
    idx = subcarrier_indices.astype(jnp.int32)
    tab = _concat(basis_real, basis_imag)
    g = _gather(idx, tab)
    n_r, n_i = _normalize(g)
    return lax.complex(n_r, n_i)
